# baseline (speedup 1.0000x reference)
import functools

import numpy as np
import jax
import jax.numpy as jnp
from jax import lax
from jax.experimental import pallas as pl
from jax.experimental.pallas import tpu as pltpu
import jax.experimental.pallas.tpu_sc as plsc

_ROWS = 128
_N = 100000

_ROT_A = (13, 15, 26, 6)
_ROT_B = (17, 29, 16, 24)


def _np_threefry2x32(k0, k1, x0, x1):
    k0 = np.uint32(k0); k1 = np.uint32(k1)
    ks2 = np.uint32(k0 ^ k1 ^ np.uint32(0x1BD11BDA))
    x0 = np.uint32(x0); x1 = np.uint32(x1)

    def rotl(x, r):
        return np.uint32((np.uint64(x) << np.uint64(r)) & np.uint64(0xFFFFFFFF)) | np.uint32(x >> np.uint32(32 - r))

    add = lambda a, b: np.uint32((np.uint64(a) + np.uint64(b)) & np.uint64(0xFFFFFFFF))

    def rounds(x0, x1, rots):
        for r in rots:
            x0 = add(x0, x1)
            x1 = rotl(x1, r)
            x1 = np.uint32(x1 ^ x0)
        return x0, x1

    x0 = add(x0, k0); x1 = add(x1, k1)
    x0, x1 = rounds(x0, x1, _ROT_A)
    x0 = add(x0, k1); x1 = add(add(x1, ks2), 1)
    x0, x1 = rounds(x0, x1, _ROT_B)
    x0 = add(x0, ks2); x1 = add(add(x1, k0), 2)
    x0, x1 = rounds(x0, x1, _ROT_A)
    x0 = add(x0, k0); x1 = add(add(x1, k1), 3)
    x0, x1 = rounds(x0, x1, _ROT_B)
    x0 = add(x0, k1); x1 = add(add(x1, ks2), 4)
    x0, x1 = rounds(x0, x1, _ROT_A)
    x0 = add(x0, ks2); x1 = add(add(x1, k0), 5)
    return x0, x1


_FK0, _FK1 = _np_threefry2x32(0, 0, 0, 1234)


def _np_uniform():
    n = _ROWS * _N
    i = np.arange(n, dtype=np.uint32)
    o0, o1 = _np_threefry2x32(_FK0, _FK1, np.zeros_like(i), i)
    bits = o0 ^ o1
    fb = (bits >> np.uint32(9)) | np.uint32(0x3F800000)
    f = fb.view(np.float32) - np.float32(1.0)
    delta = np.float32(np.float32(1.0) - np.float32(1e-8))
    u = np.maximum(np.float32(1e-8), f * delta + np.float32(1e-8))
    return u.reshape(_ROWS, _N)


_U_CONST = _np_uniform()

_C = 16384
_NB = (_N + _C - 1) // _C
_NLAST = _N - 128


def _argmax_kernel(inp_ref, u_ref, idx_ref, maxs, idxs):
    j = pl.program_id(0)

    @pl.when(j == 0)
    def _init():
        maxs[...] = jnp.full((_ROWS, 1), -jnp.inf, jnp.float32)
        idxs[...] = jnp.zeros((_ROWS, 1), jnp.int32)

    x = inp_ref[...]
    col_l = lax.broadcasted_iota(jnp.int32, (_ROWS, _C), 1)
    col_g = col_l + j * _C
    g = -jnp.log(-jnp.log(u_ref[...]))
    s = x + g
    s = jnp.where(col_g < _N, s, -jnp.inf)
    m = jnp.max(s, axis=1, keepdims=True)
    cand = jnp.where(s == m, col_l, jnp.int32(_C))
    il = jnp.min(cand, axis=1, keepdims=True)
    ig = il + j * _C
    better = m > maxs[...]
    idxs[...] = jnp.where(better, ig, idxs[...])
    maxs[...] = jnp.where(better, m, maxs[...])

    @pl.when(j == _NB - 1)
    def _fin():
        idx_ref[...] = idxs[...]


_FLAT = _ROWS * _N
_SC_TECS = 32
_SC_CHUNK = 20000
_SC_NCOPY = _FLAT // (_SC_TECS * _SC_CHUNK)

_SC_MESH = plsc.VectorSubcoreMesh(core_axis_name="c", subcore_axis_name="s")


@functools.partial(
    pl.kernel,
    out_type=jax.ShapeDtypeStruct((_FLAT,), jnp.float32),
    mesh=_SC_MESH,
    scratch_types=[
        pltpu.VMEM((_SC_CHUNK,), jnp.float32),
        pltpu.SemaphoreType.DMA,
    ],
)
def _sc_zero_kernel(out_hbm, zbuf, sem):
    tid = lax.axis_index("c") * 16 + lax.axis_index("s")

    def zero_body(i, carry):
        zbuf[pl.ds(i * 16, 16)] = jnp.zeros((16,), jnp.float32)
        return carry

    lax.fori_loop(0, _SC_CHUNK // 16, zero_body, 0)
    base = tid * (_SC_CHUNK * _SC_NCOPY)
    copies = []
    for k in range(_SC_NCOPY):
        off = pl.multiple_of(base + k * _SC_CHUNK, 8)
        cp = pltpu.make_async_copy(
            zbuf, out_hbm.at[pl.ds(off, _SC_CHUNK)], sem)
        cp.start()
        copies.append(cp)
    for cp in copies:
        cp.wait()


def _ones_kernel(zout_ref, idx_v_ref, idx_s_ref, out_ref, pat_ref, sem):
    idx_v = idx_v_ref[...]
    start_v = jnp.minimum((idx_v // 128) * 128, _NLAST)
    off_v = idx_v - start_v
    sub = lax.broadcasted_iota(jnp.int32, (_ROWS, 128), 1)
    pat_ref[...] = (sub == off_v).astype(jnp.float32)
    copies = []
    for r in range(_ROWS):
        c = idx_s_ref[r, 0]
        c128 = pl.multiple_of(jnp.minimum((c // 128) * 128, _NLAST), 128)
        cp = pltpu.make_async_copy(
            pat_ref.at[pl.ds(r, 1), :],
            out_ref.at[pl.ds(r, 1), pl.ds(c128, 128)],
            sem)
        cp.start()
        copies.append(cp)
    for cp in copies:
        cp.wait()


def kernel(inputs):
    idx = pl.pallas_call(
        _argmax_kernel,
        grid=(_NB,),
        in_specs=[
            pl.BlockSpec((_ROWS, _C), lambda j: (0, j)),
            pl.BlockSpec((_ROWS, _C), lambda j: (0, j)),
        ],
        out_specs=pl.BlockSpec((_ROWS, 1), lambda j: (0, 0)),
        out_shape=jax.ShapeDtypeStruct((_ROWS, 1), jnp.int32),
        scratch_shapes=[
            pltpu.VMEM((_ROWS, 1), jnp.float32),
            pltpu.VMEM((_ROWS, 1), jnp.int32),
        ],
    )(inputs, jnp.asarray(_U_CONST))
    zout = _sc_zero_kernel().reshape(_ROWS, _N)
    out = pl.pallas_call(
        _ones_kernel,
        in_specs=[
            pl.BlockSpec(memory_space=pl.ANY),
            pl.BlockSpec(memory_space=pltpu.VMEM),
            pl.BlockSpec(memory_space=pltpu.SMEM),
        ],
        out_specs=pl.BlockSpec(memory_space=pl.ANY),
        out_shape=jax.ShapeDtypeStruct((_ROWS, _N), jnp.float32),
        input_output_aliases={0: 0},
        scratch_shapes=[
            pltpu.VMEM((_ROWS, 128), jnp.float32),
            pltpu.SemaphoreType.DMA,
        ],
    )(zout, idx, idx)
    return out


# speedup vs baseline: 1.5813x; 1.5813x over previous
import numpy as np
import jax
import jax.numpy as jnp
from jax import lax
from jax.experimental import pallas as pl
from jax.experimental.pallas import tpu as pltpu

_ROWS = 128
_N = 100000

_ROT_A = (13, 15, 26, 6)
_ROT_B = (17, 29, 16, 24)


def _np_threefry2x32(k0, k1, x0, x1):
    k0 = np.uint32(k0); k1 = np.uint32(k1)
    ks2 = np.uint32(k0 ^ k1 ^ np.uint32(0x1BD11BDA))
    x0 = np.uint32(x0); x1 = np.uint32(x1)

    def rotl(x, r):
        return np.uint32((np.uint64(x) << np.uint64(r)) & np.uint64(0xFFFFFFFF)) | np.uint32(x >> np.uint32(32 - r))

    add = lambda a, b: np.uint32((np.uint64(a) + np.uint64(b)) & np.uint64(0xFFFFFFFF))

    def rounds(x0, x1, rots):
        for r in rots:
            x0 = add(x0, x1)
            x1 = rotl(x1, r)
            x1 = np.uint32(x1 ^ x0)
        return x0, x1

    x0 = add(x0, k0); x1 = add(x1, k1)
    x0, x1 = rounds(x0, x1, _ROT_A)
    x0 = add(x0, k1); x1 = add(add(x1, ks2), 1)
    x0, x1 = rounds(x0, x1, _ROT_B)
    x0 = add(x0, ks2); x1 = add(add(x1, k0), 2)
    x0, x1 = rounds(x0, x1, _ROT_A)
    x0 = add(x0, k0); x1 = add(add(x1, k1), 3)
    x0, x1 = rounds(x0, x1, _ROT_B)
    x0 = add(x0, k1); x1 = add(add(x1, ks2), 4)
    x0, x1 = rounds(x0, x1, _ROT_A)
    x0 = add(x0, ks2); x1 = add(add(x1, k0), 5)
    return x0, x1


_FK0, _FK1 = _np_threefry2x32(0, 0, 0, 1234)


def _np_uniform():
    n = _ROWS * _N
    i = np.arange(n, dtype=np.uint32)
    o0, o1 = _np_threefry2x32(_FK0, _FK1, np.zeros_like(i), i)
    bits = o0 ^ o1
    fb = (bits >> np.uint32(9)) | np.uint32(0x3F800000)
    f = fb.view(np.float32) - np.float32(1.0)
    delta = np.float32(np.float32(1.0) - np.float32(1e-8))
    u = np.maximum(np.float32(1e-8), f * delta + np.float32(1e-8))
    return u.reshape(_ROWS, _N)


_U_CONST = _np_uniform()

_C = 16384
_NB = (_N + _C - 1) // _C
_NLAST = _N - 128


def _argmax_kernel(inp_ref, u_ref, zout_ref, idx_ref, maxs, idxs):
    j = pl.program_id(0)

    @pl.when(j == 0)
    def _init():
        maxs[...] = jnp.full((_ROWS, 1), -jnp.inf, jnp.float32)
        idxs[...] = jnp.zeros((_ROWS, 1), jnp.int32)

    zout_ref[...] = jnp.zeros((_ROWS, _C), jnp.float32)

    x = inp_ref[...]
    col_l = lax.broadcasted_iota(jnp.int32, (_ROWS, _C), 1)
    col_g = col_l + j * _C
    g = -jnp.log(-jnp.log(u_ref[...]))
    s = x + g
    s = jnp.where(col_g < _N, s, -jnp.inf)
    m = jnp.max(s, axis=1, keepdims=True)
    cand = jnp.where(s == m, col_l, jnp.int32(_C))
    il = jnp.min(cand, axis=1, keepdims=True)
    ig = il + j * _C
    better = m > maxs[...]
    idxs[...] = jnp.where(better, ig, idxs[...])
    maxs[...] = jnp.where(better, m, maxs[...])

    @pl.when(j == _NB - 1)
    def _fin():
        idx_ref[...] = idxs[...]


def _ones_kernel(zout_ref, idx_v_ref, idx_s_ref, out_ref, pat_ref, sem):
    idx_v = idx_v_ref[...]
    start_v = jnp.minimum((idx_v // 128) * 128, _NLAST)
    off_v = idx_v - start_v
    sub = lax.broadcasted_iota(jnp.int32, (_ROWS, 128), 1)
    pat_ref[...] = (sub == off_v).astype(jnp.float32)
    copies = []
    for r in range(_ROWS):
        c = idx_s_ref[r, 0]
        c128 = pl.multiple_of(jnp.minimum((c // 128) * 128, _NLAST), 128)
        cp = pltpu.make_async_copy(
            pat_ref.at[pl.ds(r, 1), :],
            out_ref.at[pl.ds(r, 1), pl.ds(c128, 128)],
            sem)
        cp.start()
        copies.append(cp)
    for cp in copies:
        cp.wait()


def kernel(inputs):
    zout, idx = pl.pallas_call(
        _argmax_kernel,
        grid=(_NB,),
        in_specs=[
            pl.BlockSpec((_ROWS, _C), lambda j: (0, j)),
            pl.BlockSpec((_ROWS, _C), lambda j: (0, j)),
        ],
        out_specs=[
            pl.BlockSpec((_ROWS, _C), lambda j: (0, j)),
            pl.BlockSpec((_ROWS, 1), lambda j: (0, 0)),
        ],
        out_shape=[
            jax.ShapeDtypeStruct((_ROWS, _N), jnp.float32),
            jax.ShapeDtypeStruct((_ROWS, 1), jnp.int32),
        ],
        scratch_shapes=[
            pltpu.VMEM((_ROWS, 1), jnp.float32),
            pltpu.VMEM((_ROWS, 1), jnp.int32),
        ],
    )(inputs, jnp.asarray(_U_CONST))
    out = pl.pallas_call(
        _ones_kernel,
        in_specs=[
            pl.BlockSpec(memory_space=pl.ANY),
            pl.BlockSpec(memory_space=pltpu.VMEM),
            pl.BlockSpec(memory_space=pltpu.SMEM),
        ],
        out_specs=pl.BlockSpec(memory_space=pl.ANY),
        out_shape=jax.ShapeDtypeStruct((_ROWS, _N), jnp.float32),
        input_output_aliases={0: 0},
        scratch_shapes=[
            pltpu.VMEM((_ROWS, 128), jnp.float32),
            pltpu.SemaphoreType.DMA,
        ],
    )(zout, idx, idx)
    return out
